# Initial kernel scaffold; baseline (speedup 1.0000x reference)
import jax
import jax.numpy as jnp
from jax.experimental import pallas as pl


def kernel(features, edge_index, gc1_W, gc1_b, gc2_W, gc2_b, enc_W1, enc_b1, enc_W2, enc_b2, dec_W1, dec_b1, dec_W2, dec_b2, pl_W, pl_b):
    raise NotImplementedError("write your pallas kernel here")


# speedup vs baseline: 11.1816x; 11.1816x over previous
import functools

import jax
import jax.numpy as jnp
from jax import lax
from jax.experimental import pallas as pl
from jax.experimental.pallas import tpu as pltpu
from jax.experimental.pallas import tpu_sc as plsc

N = 10000
NPAD = 10112
F_IN = 128
NHID = 32
LAT = 16
NCLS = 16
E = 320000

NC = 2
NS = 16
NW = NC * NS
CHUNK = 128
NCHUNKS = -(-(E // NW) // CHUNK)
EPT = NCHUNKS * CHUNK
ROWS_PER_TILE = NPAD // NS

_HI = jax.lax.Precision.HIGHEST


def _dot(a, b):
    return lax.dot_general(a, b, (((1,), (0,)), ((), ())),
                           precision=_HI, preferred_element_type=jnp.float32)


def _sigmoid(x):
    return 1.0 / (1.0 + jnp.exp(-x))


def _sc_mesh():
    return plsc.VectorSubcoreMesh(core_axis_name="c", subcore_axis_name="s",
                                  num_cores=NC, num_subcores=NS)


def _zero_fill(buf, d):
    z = jnp.zeros((16,), jnp.float32)

    def body(i, carry):
        for c in range(d // 16):
            buf[i, pl.ds(c * 16, 16)] = z
        return carry

    lax.fori_loop(0, buf.shape[0], body, 0)


@functools.lru_cache(maxsize=None)
def _sc_degree_kernel():
    @functools.partial(
        pl.kernel,
        out_type=jax.ShapeDtypeStruct((NC, NPAD, 16), jnp.float32),
        mesh=_sc_mesh(),
        scratch_types=[
            pltpu.VMEM((CHUNK,), jnp.int32),
            pltpu.VMEM((CHUNK, 16), jnp.float32),
            pltpu.VMEM((ROWS_PER_TILE, 16), jnp.float32),
            pltpu.VMEM_SHARED((NPAD, 16), jnp.float32),
        ],
        compiler_params=pltpu.CompilerParams(use_tc_tiling_on_sc=False))
    def k(dst_hbm, out_hbm, dbuf, ones_v, zbuf, acc):
        cid = lax.axis_index("c")
        sid = lax.axis_index("s")
        wid = cid * NS + sid
        _zero_fill(zbuf, 16)
        one = jnp.ones((16,), jnp.float32)

        def fill1(i, carry):
            ones_v[i, pl.ds(0, 16)] = one
            return carry

        lax.fori_loop(0, CHUNK, fill1, 0)
        sl = pl.ds(sid * ROWS_PER_TILE, ROWS_PER_TILE)
        pltpu.sync_copy(zbuf, acc.at[sl])
        plsc.subcore_barrier()

        def step(j, carry):
            pltpu.sync_copy(dst_hbm.at[wid, j], dbuf)
            pltpu.sync_copy(ones_v, acc.at[dbuf], add=True)
            return carry

        lax.fori_loop(0, NCHUNKS, step, 0)
        plsc.subcore_barrier()
        pltpu.sync_copy(acc.at[sl], out_hbm.at[cid, sl])

    return k


@functools.lru_cache(maxsize=None)
def _make_sc_segsum(d):
    @functools.partial(
        pl.kernel,
        out_type=jax.ShapeDtypeStruct((NC, NPAD, d), jnp.float32),
        mesh=_sc_mesh(),
        scratch_types=[
            pltpu.VMEM((NCHUNKS, CHUNK), jnp.int32),
            pltpu.VMEM((CHUNK,), jnp.int32),
            pltpu.VMEM((CHUNK, d), jnp.float32),
            pltpu.VMEM((ROWS_PER_TILE, d), jnp.float32),
            pltpu.VMEM_SHARED((NPAD, d), jnp.float32),
            pltpu.SemaphoreType.DMA,
        ],
        compiler_params=pltpu.CompilerParams(use_tc_tiling_on_sc=False))
    def k(table_hbm, src_hbm, dst_hbm, out_hbm, sidx, dbuf, rows, zbuf, acc, sem):
        cid = lax.axis_index("c")
        sid = lax.axis_index("s")
        wid = cid * NS + sid
        _zero_fill(zbuf, d)
        sl = pl.ds(sid * ROWS_PER_TILE, ROWS_PER_TILE)
        pltpu.sync_copy(zbuf, acc.at[sl])
        pltpu.sync_copy(src_hbm.at[wid], sidx)
        plsc.subcore_barrier()

        def step(j, carry):
            gather = pltpu.async_copy(table_hbm.at[sidx.at[j]], rows, sem)
            pltpu.sync_copy(dst_hbm.at[wid, j], dbuf)
            gather.wait()
            pltpu.sync_copy(rows, acc.at[dbuf], add=True)
            return carry

        lax.fori_loop(0, NCHUNKS, step, 0)
        plsc.subcore_barrier()
        pltpu.sync_copy(acc.at[sl], out_hbm.at[cid, sl])

    return k


BR = 1000
BA = 400


def _full(shape):
    return pl.BlockSpec(shape, lambda i: tuple(0 for _ in shape))


def _rows(d):
    return pl.BlockSpec((BR, d), lambda i: (i, 0))


def _tc_encode(x, gc1_W, enc_W1, enc_b1, enc_W2, enc_b2,
               dec_W1, dec_b1, dec_W2, dec_b2):
    def body(x_ref, w1, ew1, eb1, ew2, eb2, dw1, db1, dw2, db2,
             h1_ref, zx_ref, fr_ref):
        x = x_ref[...]
        h1_ref[...] = _dot(x, w1[...])
        e1 = jnp.maximum(_dot(x, ew1[...]) + eb1[...], 0.0)
        zx = _dot(e1, ew2[...]) + eb2[...]
        zx_ref[...] = zx
        dh = _dot(zx, dw1[...]) + db1[...]
        fr_ref[...] = _sigmoid(_dot(dh, dw2[...]) + db2[...])

    return pl.pallas_call(
        body,
        grid=(N // BR,),
        in_specs=[_rows(F_IN),
                  _full((F_IN, NHID)), _full((F_IN, NHID)), _full((1, NHID)),
                  _full((NHID, LAT)), _full((1, LAT)),
                  _full((LAT, NHID)), _full((1, NHID)),
                  _full((NHID, F_IN)), _full((1, F_IN))],
        out_specs=[_rows(NHID), _rows(LAT), _rows(F_IN)],
        out_shape=[jax.ShapeDtypeStruct((N, NHID), jnp.float32),
                   jax.ShapeDtypeStruct((N, LAT), jnp.float32),
                   jax.ShapeDtypeStruct((N, F_IN), jnp.float32)],
    )(x, gc1_W, enc_W1, enc_b1, enc_W2, enc_b2, dec_W1, dec_b1, dec_W2, dec_b2)


def _deg_spec():
    return pl.BlockSpec((NC, BR, 16), lambda i: (0, i, 0))


def _dinv_of(dg):
    deg = 1.0 + dg[0, :, 0] + dg[1, :, 0]
    return lax.rsqrt(deg)[:, None]


def _tc_g1(deg2, h1):
    def body(dg, h1_ref, g1_ref):
        g1_ref[...] = h1_ref[...] * _dinv_of(dg[...])

    return pl.pallas_call(
        body,
        grid=(N // BR,),
        in_specs=[_deg_spec(), _rows(NHID)],
        out_specs=_rows(NHID),
        out_shape=jax.ShapeDtypeStruct((N, NHID), jnp.float32),
    )(deg2, h1)


def _tc_mid(agg1, h1, deg2, gc1_b, gc2_W):
    def body(ag, h1_ref, dg, b1, w2, h2_ref, g2_ref):
        dinv = _dinv_of(dg[...])
        a = ag[0] + ag[1]
        hmid = jnp.maximum(a * dinv + h1_ref[...] * (dinv * dinv) + b1[...], 0.0)
        h2 = _dot(hmid, w2[...])
        h2_ref[...] = h2
        g2_ref[...] = h2 * dinv

    return pl.pallas_call(
        body,
        grid=(N // BR,),
        in_specs=[pl.BlockSpec((NC, BR, NHID), lambda i: (0, i, 0)),
                  _rows(NHID), _deg_spec(), _full((1, NHID)),
                  _full((NHID, LAT))],
        out_specs=[_rows(LAT), _rows(LAT)],
        out_shape=[jax.ShapeDtypeStruct((N, LAT), jnp.float32),
                   jax.ShapeDtypeStruct((N, LAT), jnp.float32)],
    )(agg1, h1, deg2, gc1_b, gc2_W)


def _tc_head(agg2, h2, deg2, gc2_b, zx, pl_W, pl_b):
    def body(ag, h2_ref, dg, b2, zx_ref, plw, plb, z_ref, pred_ref):
        dinv = _dinv_of(dg[...])
        za = jnp.maximum((ag[0] + ag[1]) * dinv
                         + h2_ref[...] * (dinv * dinv) + b2[...], 0.0)
        z = jnp.concatenate([za, zx_ref[...]], axis=1)
        z_ref[...] = z
        lg = _dot(z, plw[...]) + plb[...]
        m = jnp.max(lg, axis=1, keepdims=True)
        e = jnp.exp(lg - m)
        pred_ref[...] = e / jnp.sum(e, axis=1, keepdims=True)

    return pl.pallas_call(
        body,
        grid=(N // BR,),
        in_specs=[pl.BlockSpec((NC, BR, LAT), lambda i: (0, i, 0)),
                  _rows(LAT), _deg_spec(), _full((1, LAT)), _rows(LAT),
                  _full((2 * LAT, NCLS)), _full((1, NCLS))],
        out_specs=[_rows(2 * LAT), _rows(NCLS)],
        out_shape=[jax.ShapeDtypeStruct((N, 2 * LAT), jnp.float32),
                   jax.ShapeDtypeStruct((N, NCLS), jnp.float32)],
    )(agg2, h2, deg2, gc2_b, zx, pl_W, pl_b)


def _tc_adj(z):
    def body(zi, zj, out_ref):
        out_ref[...] = _sigmoid(lax.dot_general(
            zi[...], zj[...], (((1,), (1,)), ((), ())),
            precision=_HI, preferred_element_type=jnp.float32))

    return pl.pallas_call(
        body,
        grid=(N // BA,),
        in_specs=[pl.BlockSpec((BA, 2 * LAT), lambda i: (i, 0)),
                  pl.BlockSpec((N, 2 * LAT), lambda i: (0, 0))],
        out_specs=pl.BlockSpec((BA, N), lambda i: (i, 0)),
        out_shape=jax.ShapeDtypeStruct((N, N), jnp.float32),
    )(z, z)


def kernel(features, edge_index, gc1_W, gc1_b, gc2_W, gc2_b,
           enc_W1, enc_b1, enc_W2, enc_b2,
           dec_W1, dec_b1, dec_W2, dec_b2, pl_W, pl_b):
    src = edge_index[0]
    dst = edge_index[1]
    pad = jnp.full((EPT * NW - E,), NPAD - 1, jnp.int32)
    src3 = jnp.concatenate([src, pad]).reshape(NW, NCHUNKS, CHUNK)
    dst3 = jnp.concatenate([dst, pad]).reshape(NW, NCHUNKS, CHUNK)
    b = lambda v: v.reshape(1, -1)

    deg2 = _sc_degree_kernel()(dst3)
    h1, zx, fr = _tc_encode(features, gc1_W, enc_W1, b(enc_b1), enc_W2,
                            b(enc_b2), dec_W1, b(dec_b1), dec_W2, b(dec_b2))
    g1 = _tc_g1(deg2, h1)
    g1p = jnp.pad(g1, ((0, NPAD - N), (0, 0)))
    agg1 = _make_sc_segsum(NHID)(g1p, src3, dst3)
    h2, g2 = _tc_mid(agg1, h1, deg2, b(gc1_b), gc2_W)
    g2p = jnp.pad(g2, ((0, NPAD - N), (0, 0)))
    agg2 = _make_sc_segsum(LAT)(g2p, src3, dst3)
    z, pred = _tc_head(agg2, h2, deg2, b(gc2_b), zx, pl_W, b(pl_b))
    adj = _tc_adj(z)
    return adj, fr, pred, z


# speedup vs baseline: 15.5549x; 1.3911x over previous
import functools

import jax
import jax.numpy as jnp
from jax import lax
from jax.experimental import pallas as pl
from jax.experimental.pallas import tpu as pltpu
from jax.experimental.pallas import tpu_sc as plsc

N = 10000
NPAD = 10112
F_IN = 128
NHID = 32
LAT = 16
NCLS = 16
E = 320000

NC = 2
NS = 16
NW = NC * NS
CHUNK = 128
NCHUNKS = -(-(E // NW) // CHUNK)
EPT = NCHUNKS * CHUNK
ROWS_PER_TILE = NPAD // NS

def _dot(a, b):
    return lax.dot_general(a, b, (((1,), (0,)), ((), ())),
                           preferred_element_type=jnp.float32)


def _sigmoid(x):
    return 1.0 / (1.0 + jnp.exp(-x))


def _sc_mesh():
    return plsc.VectorSubcoreMesh(core_axis_name="c", subcore_axis_name="s",
                                  num_cores=NC, num_subcores=NS)


def _zero_fill(buf, d):
    z = jnp.zeros((16,), jnp.float32)

    def body(i, carry):
        for c in range(d // 16):
            buf[i, pl.ds(c * 16, 16)] = z
        return carry

    lax.fori_loop(0, buf.shape[0], body, 0)


@functools.lru_cache(maxsize=None)
def _sc_degree_kernel():
    @functools.partial(
        pl.kernel,
        out_type=jax.ShapeDtypeStruct((NC, NPAD, 16), jnp.float32),
        mesh=_sc_mesh(),
        scratch_types=[
            pltpu.VMEM((CHUNK,), jnp.int32),
            pltpu.VMEM((CHUNK, 16), jnp.float32),
            pltpu.VMEM((ROWS_PER_TILE, 16), jnp.float32),
            pltpu.VMEM_SHARED((NPAD, 16), jnp.float32),
        ],
        compiler_params=pltpu.CompilerParams(use_tc_tiling_on_sc=False))
    def k(dst_hbm, out_hbm, dbuf, ones_v, zbuf, acc):
        cid = lax.axis_index("c")
        sid = lax.axis_index("s")
        wid = cid * NS + sid
        _zero_fill(zbuf, 16)
        one = jnp.ones((16,), jnp.float32)

        def fill1(i, carry):
            ones_v[i, pl.ds(0, 16)] = one
            return carry

        lax.fori_loop(0, CHUNK, fill1, 0)
        sl = pl.ds(sid * ROWS_PER_TILE, ROWS_PER_TILE)
        pltpu.sync_copy(zbuf, acc.at[sl])
        plsc.subcore_barrier()

        def step(j, carry):
            pltpu.sync_copy(dst_hbm.at[wid, j], dbuf)
            pltpu.sync_copy(ones_v, acc.at[dbuf], add=True)
            return carry

        lax.fori_loop(0, NCHUNKS, step, 0)
        plsc.subcore_barrier()
        pltpu.sync_copy(acc.at[sl], out_hbm.at[cid, sl])

    return k


@functools.lru_cache(maxsize=None)
def _make_sc_segsum(d):
    @functools.partial(
        pl.kernel,
        out_type=jax.ShapeDtypeStruct((NC, NPAD, d), jnp.float32),
        mesh=_sc_mesh(),
        scratch_types=[
            pltpu.VMEM((NCHUNKS, CHUNK), jnp.int32),
            pltpu.VMEM((CHUNK,), jnp.int32),
            pltpu.VMEM((CHUNK, d), jnp.float32),
            pltpu.VMEM((ROWS_PER_TILE, d), jnp.float32),
            pltpu.VMEM_SHARED((NPAD, d), jnp.float32),
            pltpu.SemaphoreType.DMA,
        ],
        compiler_params=pltpu.CompilerParams(use_tc_tiling_on_sc=False))
    def k(table_hbm, src_hbm, dst_hbm, out_hbm, sidx, dbuf, rows, zbuf, acc, sem):
        cid = lax.axis_index("c")
        sid = lax.axis_index("s")
        wid = cid * NS + sid
        _zero_fill(zbuf, d)
        sl = pl.ds(sid * ROWS_PER_TILE, ROWS_PER_TILE)
        pltpu.sync_copy(zbuf, acc.at[sl])
        pltpu.sync_copy(src_hbm.at[wid], sidx)
        plsc.subcore_barrier()

        def step(j, carry):
            gather = pltpu.async_copy(table_hbm.at[sidx.at[j]], rows, sem)
            pltpu.sync_copy(dst_hbm.at[wid, j], dbuf)
            gather.wait()
            pltpu.sync_copy(rows, acc.at[dbuf], add=True)
            return carry

        lax.fori_loop(0, NCHUNKS, step, 0)
        plsc.subcore_barrier()
        pltpu.sync_copy(acc.at[sl], out_hbm.at[cid, sl])

    return k


BR = 1000
BA = 400


def _full(shape):
    return pl.BlockSpec(shape, lambda i: tuple(0 for _ in shape))


def _rows(d):
    return pl.BlockSpec((BR, d), lambda i: (i, 0))


def _tc_encode(x, gc1_W, enc_W1, enc_b1, enc_W2, enc_b2,
               dec_W1, dec_b1, dec_W2, dec_b2):
    def body(x_ref, w1, ew1, eb1, ew2, eb2, dw1, db1, dw2, db2,
             h1_ref, zx_ref, fr_ref):
        x = x_ref[...]
        h1_ref[...] = _dot(x, w1[...])
        e1 = jnp.maximum(_dot(x, ew1[...]) + eb1[...], 0.0)
        zx = _dot(e1, ew2[...]) + eb2[...]
        zx_ref[...] = zx
        dh = _dot(zx, dw1[...]) + db1[...]
        fr_ref[...] = _sigmoid(_dot(dh, dw2[...]) + db2[...])

    return pl.pallas_call(
        body,
        grid=(N // BR,),
        in_specs=[_rows(F_IN),
                  _full((F_IN, NHID)), _full((F_IN, NHID)), _full((1, NHID)),
                  _full((NHID, LAT)), _full((1, LAT)),
                  _full((LAT, NHID)), _full((1, NHID)),
                  _full((NHID, F_IN)), _full((1, F_IN))],
        out_specs=[_rows(NHID), _rows(LAT), _rows(F_IN)],
        out_shape=[jax.ShapeDtypeStruct((N, NHID), jnp.float32),
                   jax.ShapeDtypeStruct((N, LAT), jnp.float32),
                   jax.ShapeDtypeStruct((N, F_IN), jnp.float32)],
    )(x, gc1_W, enc_W1, enc_b1, enc_W2, enc_b2, dec_W1, dec_b1, dec_W2, dec_b2)


def _deg_spec():
    return pl.BlockSpec((NC, BR, 16), lambda i: (0, i, 0))


def _dinv_of(dg):
    deg = 1.0 + dg[0, :, 0] + dg[1, :, 0]
    return lax.rsqrt(deg)[:, None]


def _tc_g1(deg2, h1):
    def body(dg, h1_ref, g1_ref):
        g1_ref[...] = h1_ref[...] * _dinv_of(dg[...])

    return pl.pallas_call(
        body,
        grid=(N // BR,),
        in_specs=[_deg_spec(), _rows(NHID)],
        out_specs=_rows(NHID),
        out_shape=jax.ShapeDtypeStruct((N, NHID), jnp.float32),
    )(deg2, h1)


def _tc_mid(agg1, h1, deg2, gc1_b, gc2_W):
    def body(ag, h1_ref, dg, b1, w2, h2_ref, g2_ref):
        dinv = _dinv_of(dg[...])
        a = ag[0] + ag[1]
        hmid = jnp.maximum(a * dinv + h1_ref[...] * (dinv * dinv) + b1[...], 0.0)
        h2 = _dot(hmid, w2[...])
        h2_ref[...] = h2
        g2_ref[...] = h2 * dinv

    return pl.pallas_call(
        body,
        grid=(N // BR,),
        in_specs=[pl.BlockSpec((NC, BR, NHID), lambda i: (0, i, 0)),
                  _rows(NHID), _deg_spec(), _full((1, NHID)),
                  _full((NHID, LAT))],
        out_specs=[_rows(LAT), _rows(LAT)],
        out_shape=[jax.ShapeDtypeStruct((N, LAT), jnp.float32),
                   jax.ShapeDtypeStruct((N, LAT), jnp.float32)],
    )(agg1, h1, deg2, gc1_b, gc2_W)


def _tc_head(agg2, h2, deg2, gc2_b, zx, pl_W, pl_b):
    def body(ag, h2_ref, dg, b2, zx_ref, plw, plb, z_ref, pred_ref):
        dinv = _dinv_of(dg[...])
        za = jnp.maximum((ag[0] + ag[1]) * dinv
                         + h2_ref[...] * (dinv * dinv) + b2[...], 0.0)
        z = jnp.concatenate([za, zx_ref[...]], axis=1)
        z_ref[...] = z
        lg = _dot(z, plw[...]) + plb[...]
        m = jnp.max(lg, axis=1, keepdims=True)
        e = jnp.exp(lg - m)
        pred_ref[...] = e / jnp.sum(e, axis=1, keepdims=True)

    return pl.pallas_call(
        body,
        grid=(N // BR,),
        in_specs=[pl.BlockSpec((NC, BR, LAT), lambda i: (0, i, 0)),
                  _rows(LAT), _deg_spec(), _full((1, LAT)), _rows(LAT),
                  _full((2 * LAT, NCLS)), _full((1, NCLS))],
        out_specs=[_rows(2 * LAT), _rows(NCLS)],
        out_shape=[jax.ShapeDtypeStruct((N, 2 * LAT), jnp.float32),
                   jax.ShapeDtypeStruct((N, NCLS), jnp.float32)],
    )(agg2, h2, deg2, gc2_b, zx, pl_W, pl_b)


def _tc_adj(z):
    def body(zi, zj, out_ref):
        out_ref[...] = _sigmoid(lax.dot_general(
            zi[...], zj[...], (((1,), (1,)), ((), ())),
            preferred_element_type=jnp.float32))

    return pl.pallas_call(
        body,
        grid=(N // BA,),
        in_specs=[pl.BlockSpec((BA, 2 * LAT), lambda i: (i, 0)),
                  pl.BlockSpec((N, 2 * LAT), lambda i: (0, 0))],
        out_specs=pl.BlockSpec((BA, N), lambda i: (i, 0)),
        out_shape=jax.ShapeDtypeStruct((N, N), jnp.float32),
    )(z, z)


def kernel(features, edge_index, gc1_W, gc1_b, gc2_W, gc2_b,
           enc_W1, enc_b1, enc_W2, enc_b2,
           dec_W1, dec_b1, dec_W2, dec_b2, pl_W, pl_b):
    src = edge_index[0]
    dst = edge_index[1]
    pad = jnp.full((EPT * NW - E,), NPAD - 1, jnp.int32)
    src3 = jnp.concatenate([src, pad]).reshape(NW, NCHUNKS, CHUNK)
    dst3 = jnp.concatenate([dst, pad]).reshape(NW, NCHUNKS, CHUNK)
    b = lambda v: v.reshape(1, -1)

    deg2 = _sc_degree_kernel()(dst3)
    h1, zx, fr = _tc_encode(features, gc1_W, enc_W1, b(enc_b1), enc_W2,
                            b(enc_b2), dec_W1, b(dec_b1), dec_W2, b(dec_b2))
    g1 = _tc_g1(deg2, h1)
    g1p = jnp.pad(g1, ((0, NPAD - N), (0, 0)))
    agg1 = _make_sc_segsum(NHID)(g1p, src3, dst3)
    h2, g2 = _tc_mid(agg1, h1, deg2, b(gc1_b), gc2_W)
    g2p = jnp.pad(g2, ((0, NPAD - N), (0, 0)))
    agg2 = _make_sc_segsum(LAT)(g2p, src3, dst3)
    z, pred = _tc_head(agg2, h2, deg2, b(gc2_b), zx, pl_W, b(pl_b))
    adj = _tc_adj(z)
    return adj, fr, pred, z


# speedup vs baseline: 16.7710x; 1.0782x over previous
import functools

import jax
import jax.numpy as jnp
from jax import lax
from jax.experimental import pallas as pl
from jax.experimental.pallas import tpu as pltpu
from jax.experimental.pallas import tpu_sc as plsc

N = 10000
NPAD = 10112
F_IN = 128
NHID = 32
LAT = 16
NCLS = 16
E = 320000

NC = 2
NS = 16
NW = NC * NS
CHUNK = 128
NBUF = 4
GROUP = 2 * NBUF
NCHUNKS = 80
NGROUPS = NCHUNKS // GROUP
EPT = NCHUNKS * CHUNK
ROWS_PER_TILE = NPAD // NS

def _dot(a, b):
    return lax.dot_general(a, b, (((1,), (0,)), ((), ())),
                           preferred_element_type=jnp.float32)


def _sigmoid(x):
    return 1.0 / (1.0 + jnp.exp(-x))


def _sc_mesh():
    return plsc.VectorSubcoreMesh(core_axis_name="c", subcore_axis_name="s",
                                  num_cores=NC, num_subcores=NS)


def _zero_fill(buf, d):
    z = jnp.zeros((16,), jnp.float32)

    def body(i, carry):
        for c in range(d // 16):
            buf[i, pl.ds(c * 16, 16)] = z
        return carry

    lax.fori_loop(0, buf.shape[0], body, 0)


def _copy_idx(didx, c, dbuf):
    for kk in range(CHUNK // 16):
        dbuf[pl.ds(kk * 16, 16)] = didx[c, pl.ds(kk * 16, 16)]


@functools.lru_cache(maxsize=None)
def _sc_degree_kernel():
    @functools.partial(
        pl.kernel,
        out_type=jax.ShapeDtypeStruct((NC, NPAD, 16), jnp.float32),
        mesh=_sc_mesh(),
        scratch_types=(
            [pltpu.VMEM((NCHUNKS, CHUNK), jnp.int32),
             pltpu.VMEM((CHUNK, 16), jnp.float32),
             pltpu.VMEM((ROWS_PER_TILE, 16), jnp.float32),
             pltpu.VMEM_SHARED((NPAD, 16), jnp.float32)]
            + [pltpu.VMEM((CHUNK,), jnp.int32) for _ in range(GROUP)]
            + [pltpu.SemaphoreType.DMA for _ in range(GROUP)]),
        compiler_params=pltpu.CompilerParams(use_tc_tiling_on_sc=False))
    def k(dst_hbm, out_hbm, didx, ones_v, zbuf, acc, *ring):
        dbufs = ring[:GROUP]
        sems = ring[GROUP:]
        cid = lax.axis_index("c")
        sid = lax.axis_index("s")
        wid = cid * NS + sid
        _zero_fill(zbuf, 16)
        one = jnp.ones((16,), jnp.float32)

        def fill1(i, carry):
            ones_v[i, pl.ds(0, 16)] = one
            return carry

        lax.fori_loop(0, CHUNK, fill1, 0)
        sl = pl.ds(sid * ROWS_PER_TILE, ROWS_PER_TILE)
        pltpu.sync_copy(zbuf, acc.at[sl])
        pltpu.sync_copy(dst_hbm.at[wid], didx)
        plsc.subcore_barrier()

        def body(t, carry):
            @pl.when(t > 0)
            def _():
                for b in range(GROUP):
                    pltpu.make_async_copy(ones_v, acc.at[dbufs[b]], sems[b]).wait()

            for b in range(GROUP):
                _copy_idx(didx, t * GROUP + b, dbufs[b])
                pltpu.async_copy(ones_v, acc.at[dbufs[b]], sems[b], add=True)
            return carry

        lax.fori_loop(0, NGROUPS, body, 0)
        for b in range(GROUP):
            pltpu.make_async_copy(ones_v, acc.at[dbufs[b]], sems[b]).wait()
        plsc.subcore_barrier()
        pltpu.sync_copy(acc.at[sl], out_hbm.at[cid, sl])

    return k


@functools.lru_cache(maxsize=None)
def _make_sc_segsum(d):
    @functools.partial(
        pl.kernel,
        out_type=jax.ShapeDtypeStruct((NC, NPAD, d), jnp.float32),
        mesh=_sc_mesh(),
        scratch_types=(
            [pltpu.VMEM((NCHUNKS, CHUNK), jnp.int32),
             pltpu.VMEM((NCHUNKS, CHUNK), jnp.int32),
             pltpu.VMEM((ROWS_PER_TILE, d), jnp.float32),
             pltpu.VMEM_SHARED((NPAD, d), jnp.float32)]
            + [pltpu.VMEM((CHUNK, d), jnp.float32) for _ in range(GROUP)]
            + [pltpu.VMEM((CHUNK,), jnp.int32) for _ in range(GROUP)]
            + [pltpu.SemaphoreType.DMA for _ in range(2 * GROUP)]),
        compiler_params=pltpu.CompilerParams(use_tc_tiling_on_sc=False))
    def k(table_hbm, src_hbm, dst_hbm, out_hbm, sidx, didx, zbuf, acc, *ring):
        rows = ring[:GROUP]
        dbufs = ring[GROUP:2 * GROUP]
        gsems = ring[2 * GROUP:3 * GROUP]
        ssems = ring[3 * GROUP:]
        cid = lax.axis_index("c")
        sid = lax.axis_index("s")
        wid = cid * NS + sid
        _zero_fill(zbuf, d)
        sl = pl.ds(sid * ROWS_PER_TILE, ROWS_PER_TILE)
        pltpu.sync_copy(zbuf, acc.at[sl])
        pltpu.sync_copy(src_hbm.at[wid], sidx)
        pltpu.sync_copy(dst_hbm.at[wid], didx)
        plsc.subcore_barrier()

        def gather(c, b):
            pltpu.async_copy(table_hbm.at[sidx.at[c]], rows[b], gsems[b])

        def gwait(c, b):
            pltpu.make_async_copy(table_hbm.at[sidx.at[c]], rows[b],
                                  gsems[b]).wait()

        def scat(b):
            pltpu.async_copy(rows[b], acc.at[dbufs[b]], ssems[b], add=True)

        def swait(b):
            pltpu.make_async_copy(rows[b], acc.at[dbufs[b]], ssems[b]).wait()

        def body(t, carry):
            base = t * GROUP
            for b in range(NBUF):
                gather(base + b, b)

            @pl.when(t > 0)
            def _():
                for b in range(NBUF, GROUP):
                    swait(b)

            for b in range(NBUF):
                _copy_idx(didx, base + b, dbufs[b])
                gwait(base + b, b)
                scat(b)
            for b in range(NBUF, GROUP):
                gather(base + b, b)
            for b in range(NBUF):
                swait(b)
            for b in range(NBUF, GROUP):
                _copy_idx(didx, base + b, dbufs[b])
                gwait(base + b, b)
                scat(b)
            return carry

        lax.fori_loop(0, NGROUPS, body, 0)
        for b in range(NBUF, GROUP):
            swait(b)
        plsc.subcore_barrier()
        pltpu.sync_copy(acc.at[sl], out_hbm.at[cid, sl])

    return k


BR = 1000
BA = 400


def _full(shape):
    return pl.BlockSpec(shape, lambda i: tuple(0 for _ in shape))


def _rows(d):
    return pl.BlockSpec((BR, d), lambda i: (i, 0))


def _tc_encode(x, gc1_W, enc_W1, enc_b1, enc_W2, enc_b2,
               dec_W1, dec_b1, dec_W2, dec_b2):
    def body(x_ref, w1, ew1, eb1, ew2, eb2, dw1, db1, dw2, db2,
             h1_ref, zx_ref, fr_ref):
        x = x_ref[...]
        h1_ref[...] = _dot(x, w1[...])
        e1 = jnp.maximum(_dot(x, ew1[...]) + eb1[...], 0.0)
        zx = _dot(e1, ew2[...]) + eb2[...]
        zx_ref[...] = zx
        dh = _dot(zx, dw1[...]) + db1[...]
        fr_ref[...] = _sigmoid(_dot(dh, dw2[...]) + db2[...])

    return pl.pallas_call(
        body,
        grid=(N // BR,),
        in_specs=[_rows(F_IN),
                  _full((F_IN, NHID)), _full((F_IN, NHID)), _full((1, NHID)),
                  _full((NHID, LAT)), _full((1, LAT)),
                  _full((LAT, NHID)), _full((1, NHID)),
                  _full((NHID, F_IN)), _full((1, F_IN))],
        out_specs=[_rows(NHID), _rows(LAT), _rows(F_IN)],
        out_shape=[jax.ShapeDtypeStruct((N, NHID), jnp.float32),
                   jax.ShapeDtypeStruct((N, LAT), jnp.float32),
                   jax.ShapeDtypeStruct((N, F_IN), jnp.float32)],
    )(x, gc1_W, enc_W1, enc_b1, enc_W2, enc_b2, dec_W1, dec_b1, dec_W2, dec_b2)


def _deg_spec():
    return pl.BlockSpec((NC, BR, 16), lambda i: (0, i, 0))


def _dinv_of(dg):
    deg = 1.0 + dg[0, :, 0] + dg[1, :, 0]
    return lax.rsqrt(deg)[:, None]


def _tc_g1(deg2, h1):
    def body(dg, h1_ref, g1_ref):
        g1_ref[...] = h1_ref[...] * _dinv_of(dg[...])

    return pl.pallas_call(
        body,
        grid=(N // BR,),
        in_specs=[_deg_spec(), _rows(NHID)],
        out_specs=_rows(NHID),
        out_shape=jax.ShapeDtypeStruct((N, NHID), jnp.float32),
    )(deg2, h1)


def _tc_mid(agg1, h1, deg2, gc1_b, gc2_W):
    def body(ag, h1_ref, dg, b1, w2, h2_ref, g2_ref):
        dinv = _dinv_of(dg[...])
        a = ag[0] + ag[1]
        hmid = jnp.maximum(a * dinv + h1_ref[...] * (dinv * dinv) + b1[...], 0.0)
        h2 = _dot(hmid, w2[...])
        h2_ref[...] = h2
        g2_ref[...] = h2 * dinv

    return pl.pallas_call(
        body,
        grid=(N // BR,),
        in_specs=[pl.BlockSpec((NC, BR, NHID), lambda i: (0, i, 0)),
                  _rows(NHID), _deg_spec(), _full((1, NHID)),
                  _full((NHID, LAT))],
        out_specs=[_rows(LAT), _rows(LAT)],
        out_shape=[jax.ShapeDtypeStruct((N, LAT), jnp.float32),
                   jax.ShapeDtypeStruct((N, LAT), jnp.float32)],
    )(agg1, h1, deg2, gc1_b, gc2_W)


def _tc_head(agg2, h2, deg2, gc2_b, zx, pl_W, pl_b):
    def body(ag, h2_ref, dg, b2, zx_ref, plw, plb, z_ref, pred_ref):
        dinv = _dinv_of(dg[...])
        za = jnp.maximum((ag[0] + ag[1]) * dinv
                         + h2_ref[...] * (dinv * dinv) + b2[...], 0.0)
        z = jnp.concatenate([za, zx_ref[...]], axis=1)
        z_ref[...] = z
        lg = _dot(z, plw[...]) + plb[...]
        m = jnp.max(lg, axis=1, keepdims=True)
        e = jnp.exp(lg - m)
        pred_ref[...] = e / jnp.sum(e, axis=1, keepdims=True)

    return pl.pallas_call(
        body,
        grid=(N // BR,),
        in_specs=[pl.BlockSpec((NC, BR, LAT), lambda i: (0, i, 0)),
                  _rows(LAT), _deg_spec(), _full((1, LAT)), _rows(LAT),
                  _full((2 * LAT, NCLS)), _full((1, NCLS))],
        out_specs=[_rows(2 * LAT), _rows(NCLS)],
        out_shape=[jax.ShapeDtypeStruct((N, 2 * LAT), jnp.float32),
                   jax.ShapeDtypeStruct((N, NCLS), jnp.float32)],
    )(agg2, h2, deg2, gc2_b, zx, pl_W, pl_b)


def _tc_adj(z):
    def body(zi, zj, out_ref):
        out_ref[...] = _sigmoid(lax.dot_general(
            zi[...], zj[...], (((1,), (1,)), ((), ())),
            preferred_element_type=jnp.float32))

    return pl.pallas_call(
        body,
        grid=(N // BA,),
        in_specs=[pl.BlockSpec((BA, 2 * LAT), lambda i: (i, 0)),
                  pl.BlockSpec((N, 2 * LAT), lambda i: (0, 0))],
        out_specs=pl.BlockSpec((BA, N), lambda i: (i, 0)),
        out_shape=jax.ShapeDtypeStruct((N, N), jnp.float32),
    )(z, z)


def kernel(features, edge_index, gc1_W, gc1_b, gc2_W, gc2_b,
           enc_W1, enc_b1, enc_W2, enc_b2,
           dec_W1, dec_b1, dec_W2, dec_b2, pl_W, pl_b):
    src = edge_index[0]
    dst = edge_index[1]
    pad = jnp.full((EPT * NW - E,), NPAD - 1, jnp.int32)
    src3 = jnp.concatenate([src, pad]).reshape(NW, NCHUNKS, CHUNK)
    dst3 = jnp.concatenate([dst, pad]).reshape(NW, NCHUNKS, CHUNK)
    b = lambda v: v.reshape(1, -1)

    deg2 = _sc_degree_kernel()(dst3)
    h1, zx, fr = _tc_encode(features, gc1_W, enc_W1, b(enc_b1), enc_W2,
                            b(enc_b2), dec_W1, b(dec_b1), dec_W2, b(dec_b2))
    g1 = _tc_g1(deg2, h1)
    g1p = jnp.pad(g1, ((0, NPAD - N), (0, 0)))
    agg1 = _make_sc_segsum(NHID)(g1p, src3, dst3)
    h2, g2 = _tc_mid(agg1, h1, deg2, b(gc1_b), gc2_W)
    g2p = jnp.pad(g2, ((0, NPAD - N), (0, 0)))
    agg2 = _make_sc_segsum(LAT)(g2p, src3, dst3)
    z, pred = _tc_head(agg2, h2, deg2, b(gc2_b), zx, pl_W, b(pl_b))
    adj = _tc_adj(z)
    return adj, fr, pred, z


# speedup vs baseline: 24.0747x; 1.4355x over previous
import functools

import jax
import jax.numpy as jnp
from jax import lax
from jax.experimental import pallas as pl
from jax.experimental.pallas import tpu as pltpu
from jax.experimental.pallas import tpu_sc as plsc

N = 10000
NPAD = 10112
F_IN = 128
NHID = 32
LAT = 16
NCLS = 16
E = 320000

NC = 2
NS = 16
NW = NC * NS
CHUNK = 128
NBUF = 4
GROUP = 2 * NBUF
NCHUNKS = 80
NGROUPS = NCHUNKS // GROUP
EPT = NCHUNKS * CHUNK
ROWS_PER_TILE = NPAD // NS

def _dot(a, b):
    return lax.dot_general(a, b, (((1,), (0,)), ((), ())),
                           preferred_element_type=jnp.float32)


def _sigmoid(x):
    return 1.0 / (1.0 + jnp.exp(-x))


def _sc_mesh():
    return plsc.VectorSubcoreMesh(core_axis_name="c", subcore_axis_name="s",
                                  num_cores=NC, num_subcores=NS)


def _zero_fill(buf, d):
    z = jnp.zeros((16,), jnp.float32)

    def body(i, carry):
        for c in range(d // 16):
            buf[i, pl.ds(c * 16, 16)] = z
        return carry

    lax.fori_loop(0, buf.shape[0], body, 0)


def _copy_idx(didx, c, dbuf):
    for kk in range(CHUNK // 16):
        dbuf[pl.ds(kk * 16, 16)] = didx[c, pl.ds(kk * 16, 16)]


@functools.lru_cache(maxsize=None)
def _sc_degree_kernel():
    @functools.partial(
        pl.kernel,
        out_type=jax.ShapeDtypeStruct((NC, NPAD, 16), jnp.float32),
        mesh=_sc_mesh(),
        scratch_types=(
            [pltpu.VMEM((NCHUNKS, CHUNK), jnp.int32),
             pltpu.VMEM((CHUNK, 16), jnp.float32),
             pltpu.VMEM((ROWS_PER_TILE, 16), jnp.float32),
             pltpu.VMEM_SHARED((NPAD, 16), jnp.float32)]
            + [pltpu.VMEM((CHUNK,), jnp.int32) for _ in range(GROUP)]
            + [pltpu.SemaphoreType.DMA for _ in range(GROUP)]),
        compiler_params=pltpu.CompilerParams(use_tc_tiling_on_sc=False))
    def k(dst_hbm, out_hbm, didx, ones_v, zbuf, acc, *ring):
        dbufs = ring[:GROUP]
        sems = ring[GROUP:]
        cid = lax.axis_index("c")
        sid = lax.axis_index("s")
        wid = cid * NS + sid
        _zero_fill(zbuf, 16)
        one = jnp.ones((16,), jnp.float32)

        def fill1(i, carry):
            ones_v[i, pl.ds(0, 16)] = one
            return carry

        lax.fori_loop(0, CHUNK, fill1, 0)
        sl = pl.ds(sid * ROWS_PER_TILE, ROWS_PER_TILE)
        pltpu.sync_copy(zbuf, acc.at[sl])
        pltpu.sync_copy(dst_hbm.at[wid], didx)
        plsc.subcore_barrier()

        def body(t, carry):
            @pl.when(t > 0)
            def _():
                for b in range(GROUP):
                    pltpu.make_async_copy(ones_v, acc.at[dbufs[b]], sems[b]).wait()

            for b in range(GROUP):
                _copy_idx(didx, t * GROUP + b, dbufs[b])
                pltpu.async_copy(ones_v, acc.at[dbufs[b]], sems[b], add=True)
            return carry

        lax.fori_loop(0, NGROUPS, body, 0)
        for b in range(GROUP):
            pltpu.make_async_copy(ones_v, acc.at[dbufs[b]], sems[b]).wait()
        plsc.subcore_barrier()
        pltpu.sync_copy(acc.at[sl], out_hbm.at[cid, sl])

    return k


@functools.lru_cache(maxsize=None)
def _make_sc_segsum(d):
    @functools.partial(
        pl.kernel,
        out_type=jax.ShapeDtypeStruct((NC, NPAD, d), jnp.float32),
        mesh=_sc_mesh(),
        scratch_types=(
            [pltpu.VMEM((NCHUNKS, CHUNK), jnp.int32),
             pltpu.VMEM((NCHUNKS, CHUNK), jnp.int32),
             pltpu.VMEM((ROWS_PER_TILE, d), jnp.float32),
             pltpu.VMEM_SHARED((NPAD, d), jnp.float32)]
            + [pltpu.VMEM((CHUNK, d), jnp.float32) for _ in range(GROUP)]
            + [pltpu.VMEM((CHUNK,), jnp.int32) for _ in range(GROUP)]
            + [pltpu.SemaphoreType.DMA for _ in range(2 * GROUP)]),
        compiler_params=pltpu.CompilerParams(use_tc_tiling_on_sc=False))
    def k(table_hbm, src_hbm, dst_hbm, out_hbm, sidx, didx, zbuf, acc, *ring):
        rows = ring[:GROUP]
        dbufs = ring[GROUP:2 * GROUP]
        gsems = ring[2 * GROUP:3 * GROUP]
        ssems = ring[3 * GROUP:]
        cid = lax.axis_index("c")
        sid = lax.axis_index("s")
        wid = cid * NS + sid
        _zero_fill(zbuf, d)
        sl = pl.ds(sid * ROWS_PER_TILE, ROWS_PER_TILE)
        pltpu.sync_copy(zbuf, acc.at[sl])
        pltpu.sync_copy(src_hbm.at[wid], sidx)
        pltpu.sync_copy(dst_hbm.at[wid], didx)
        plsc.subcore_barrier()

        def gather(c, b):
            pltpu.async_copy(table_hbm.at[sidx.at[c]], rows[b], gsems[b])

        def gwait(c, b):
            pltpu.make_async_copy(table_hbm.at[sidx.at[c]], rows[b],
                                  gsems[b]).wait()

        def scat(b):
            pltpu.async_copy(rows[b], acc.at[dbufs[b]], ssems[b], add=True)

        def swait(b):
            pltpu.make_async_copy(rows[b], acc.at[dbufs[b]], ssems[b]).wait()

        def body(t, carry):
            base = t * GROUP
            for b in range(NBUF):
                gather(base + b, b)

            @pl.when(t > 0)
            def _():
                for b in range(NBUF, GROUP):
                    swait(b)

            for b in range(NBUF):
                _copy_idx(didx, base + b, dbufs[b])
                gwait(base + b, b)
                scat(b)
            for b in range(NBUF, GROUP):
                gather(base + b, b)
            for b in range(NBUF):
                swait(b)
            for b in range(NBUF, GROUP):
                _copy_idx(didx, base + b, dbufs[b])
                gwait(base + b, b)
                scat(b)
            return carry

        lax.fori_loop(0, NGROUPS, body, 0)
        for b in range(NBUF, GROUP):
            swait(b)
        plsc.subcore_barrier()
        pltpu.sync_copy(acc.at[sl], out_hbm.at[cid, sl])

    return k


BR = 1000
BA = 400


def _full(shape):
    return pl.BlockSpec(shape, lambda i: tuple(0 for _ in shape))


def _rows(d):
    return pl.BlockSpec((BR, d), lambda i: (i, 0))


def _tc_encode(x, gc1_W, enc_W1, enc_b1, enc_W2, enc_b2,
               dec_W1, dec_b1, dec_W2, dec_b2):
    def body(x_ref, w1, ew1, eb1, ew2, eb2, dw1, db1, dw2, db2,
             h1_ref, zx_ref, fr_ref):
        x = x_ref[...]
        h1_ref[...] = _dot(x, w1[...])
        e1 = jnp.maximum(_dot(x, ew1[...]) + eb1[...], 0.0)
        zx = _dot(e1, ew2[...]) + eb2[...]
        zx_ref[...] = zx
        dh = _dot(zx, dw1[...]) + db1[...]
        fr_ref[...] = _sigmoid(_dot(dh, dw2[...]) + db2[...])

    return pl.pallas_call(
        body,
        grid=(N // BR,),
        in_specs=[_rows(F_IN),
                  _full((F_IN, NHID)), _full((F_IN, NHID)), _full((1, NHID)),
                  _full((NHID, LAT)), _full((1, LAT)),
                  _full((LAT, NHID)), _full((1, NHID)),
                  _full((NHID, F_IN)), _full((1, F_IN))],
        out_specs=[_rows(NHID), _rows(LAT), _rows(F_IN)],
        out_shape=[jax.ShapeDtypeStruct((N, NHID), jnp.float32),
                   jax.ShapeDtypeStruct((N, LAT), jnp.float32),
                   jax.ShapeDtypeStruct((N, F_IN), jnp.float32)],
    )(x, gc1_W, enc_W1, enc_b1, enc_W2, enc_b2, dec_W1, dec_b1, dec_W2, dec_b2)


def _deg_spec():
    return pl.BlockSpec((NC, BR, 16), lambda i: (0, i, 0))


def _dinv_of(dg):
    deg = 1.0 + dg[0, :, 0] + dg[1, :, 0]
    return lax.rsqrt(deg)[:, None]


def _tc_g1(deg2, h1):
    def body(dg, h1_ref, g1_ref):
        g1_ref[...] = h1_ref[...] * _dinv_of(dg[...])

    return pl.pallas_call(
        body,
        grid=(N // BR,),
        in_specs=[_deg_spec(), _rows(NHID)],
        out_specs=_rows(NHID),
        out_shape=jax.ShapeDtypeStruct((N, NHID), jnp.float32),
    )(deg2, h1)


def _tc_mid(agg1, h1, deg2, gc1_b, gc2_W):
    def body(ag, h1_ref, dg, b1, w2, h2_ref, g2_ref):
        dinv = _dinv_of(dg[...])
        a = ag[0] + ag[1]
        hmid = jnp.maximum(a * dinv + h1_ref[...] * (dinv * dinv) + b1[...], 0.0)
        h2 = _dot(hmid, w2[...])
        h2_ref[...] = h2
        g2_ref[...] = h2 * dinv

    return pl.pallas_call(
        body,
        grid=(N // BR,),
        in_specs=[pl.BlockSpec((NC, BR, NHID), lambda i: (0, i, 0)),
                  _rows(NHID), _deg_spec(), _full((1, NHID)),
                  _full((NHID, LAT))],
        out_specs=[_rows(LAT), _rows(LAT)],
        out_shape=[jax.ShapeDtypeStruct((N, LAT), jnp.float32),
                   jax.ShapeDtypeStruct((N, LAT), jnp.float32)],
    )(agg1, h1, deg2, gc1_b, gc2_W)


def _tc_head(agg2, h2, deg2, gc2_b, zx, pl_W, pl_b):
    def body(ag, h2_ref, dg, b2, zx_ref, plw, plb, z_ref, pred_ref):
        dinv = _dinv_of(dg[...])
        za = jnp.maximum((ag[0] + ag[1]) * dinv
                         + h2_ref[...] * (dinv * dinv) + b2[...], 0.0)
        z = jnp.concatenate([za, zx_ref[...]], axis=1)
        z_ref[...] = z
        lg = _dot(z, plw[...]) + plb[...]
        m = jnp.max(lg, axis=1, keepdims=True)
        e = jnp.exp(lg - m)
        pred_ref[...] = e / jnp.sum(e, axis=1, keepdims=True)

    return pl.pallas_call(
        body,
        grid=(N // BR,),
        in_specs=[pl.BlockSpec((NC, BR, LAT), lambda i: (0, i, 0)),
                  _rows(LAT), _deg_spec(), _full((1, LAT)), _rows(LAT),
                  _full((2 * LAT, NCLS)), _full((1, NCLS))],
        out_specs=[_rows(2 * LAT), _rows(NCLS)],
        out_shape=[jax.ShapeDtypeStruct((N, 2 * LAT), jnp.float32),
                   jax.ShapeDtypeStruct((N, NCLS), jnp.float32)],
    )(agg2, h2, deg2, gc2_b, zx, pl_W, pl_b)


def _tc_adj(z):
    def body(zi, zj, out_ref):
        out_ref[...] = _sigmoid(lax.dot_general(
            zi[...], zj[...], (((1,), (1,)), ((), ())),
            preferred_element_type=jnp.float32))

    return pl.pallas_call(
        body,
        grid=(N // BA,),
        in_specs=[pl.BlockSpec((BA, 2 * LAT), lambda i: (i, 0)),
                  pl.BlockSpec((N, 2 * LAT), lambda i: (0, 0))],
        out_specs=pl.BlockSpec((BA, N), lambda i: (i, 0)),
        out_shape=jax.ShapeDtypeStruct((N, N), jnp.float32),
    )(z, z)


def kernel(features, edge_index, gc1_W, gc1_b, gc2_W, gc2_b,
           enc_W1, enc_b1, enc_W2, enc_b2,
           dec_W1, dec_b1, dec_W2, dec_b2, pl_W, pl_b):
    src = edge_index[0]
    dst = edge_index[1]
    ar = jnp.arange(EPT * NW - E, dtype=jnp.int32)
    pad_idx = N + (ar % (NPAD - N))
    src3 = jnp.concatenate([src, pad_idx]).reshape(NW, NCHUNKS, CHUNK)
    dst3 = jnp.concatenate([dst, pad_idx]).reshape(NW, NCHUNKS, CHUNK)
    b = lambda v: v.reshape(1, -1)

    deg2 = _sc_degree_kernel()(dst3)
    h1, zx, fr = _tc_encode(features, gc1_W, enc_W1, b(enc_b1), enc_W2,
                            b(enc_b2), dec_W1, b(dec_b1), dec_W2, b(dec_b2))
    g1 = _tc_g1(deg2, h1)
    g1p = jnp.pad(g1, ((0, NPAD - N), (0, 0)))
    agg1 = _make_sc_segsum(NHID)(g1p, src3, dst3)
    h2, g2 = _tc_mid(agg1, h1, deg2, b(gc1_b), gc2_W)
    g2p = jnp.pad(g2, ((0, NPAD - N), (0, 0)))
    agg2 = _make_sc_segsum(LAT)(g2p, src3, dst3)
    z, pred = _tc_head(agg2, h2, deg2, b(gc2_b), zx, pl_W, b(pl_b))
    adj = _tc_adj(z)
    return adj, fr, pred, z


# speedup vs baseline: 25.3887x; 1.0546x over previous
import functools

import jax
import jax.numpy as jnp
from jax import lax
from jax.experimental import pallas as pl
from jax.experimental.pallas import tpu as pltpu
from jax.experimental.pallas import tpu_sc as plsc

N = 10000
NPAD = 10112
F_IN = 128
NHID = 32
LAT = 16
NCLS = 16
E = 320000

NC = 2
NS = 16
NW = NC * NS
CHUNK = 128
NBUF = 4
GROUP = 2 * NBUF
NCHUNKS = 80
NGROUPS = NCHUNKS // GROUP
EPT = NCHUNKS * CHUNK
EPW = E // NW
ROWS_PER_TILE = NPAD // NS

def _dot(a, b):
    return lax.dot_general(a, b, (((1,), (0,)), ((), ())),
                           preferred_element_type=jnp.float32)


def _sigmoid(x):
    return lax.logistic(x)


def _sc_mesh():
    return plsc.VectorSubcoreMesh(core_axis_name="c", subcore_axis_name="s",
                                  num_cores=NC, num_subcores=NS)


def _zero_fill(buf, d):
    z = jnp.zeros((16,), jnp.float32)

    def body(i, carry):
        for c in range(d // 16):
            buf[i, pl.ds(c * 16, 16)] = z
        return carry

    lax.fori_loop(0, buf.shape[0], body, 0)


def _copy_idx(idxbuf, c, dbuf):
    for kk in range(CHUNK // 16):
        dbuf[pl.ds(kk * 16, 16)] = idxbuf[pl.ds(c * CHUNK + kk * 16, 16)]


def _load_edges(ei_hbm, row, wid, idxbuf):
    pltpu.sync_copy(ei_hbm.at[row, pl.ds(wid * EPW, EPW)],
                    idxbuf.at[pl.ds(0, EPW)])
    lanes = lax.iota(jnp.int32, 16)
    for t in range(EPW, EPT, 16):
        idxbuf[pl.ds(t, 16)] = N + ((t - EPW + lanes) % (NPAD - N))


@functools.lru_cache(maxsize=None)
def _sc_degree_kernel():
    @functools.partial(
        pl.kernel,
        out_type=jax.ShapeDtypeStruct((NC, NPAD, 16), jnp.float32),
        mesh=_sc_mesh(),
        scratch_types=(
            [pltpu.VMEM((EPT,), jnp.int32),
             pltpu.VMEM((CHUNK, 16), jnp.float32),
             pltpu.VMEM((ROWS_PER_TILE, 16), jnp.float32),
             pltpu.VMEM_SHARED((NPAD, 16), jnp.float32)]
            + [pltpu.VMEM((CHUNK,), jnp.int32) for _ in range(GROUP)]
            + [pltpu.SemaphoreType.DMA for _ in range(GROUP)]),
        compiler_params=pltpu.CompilerParams(use_tc_tiling_on_sc=False))
    def k(ei_hbm, out_hbm, didx, ones_v, zbuf, acc, *ring):
        dbufs = ring[:GROUP]
        sems = ring[GROUP:]
        cid = lax.axis_index("c")
        sid = lax.axis_index("s")
        wid = cid * NS + sid
        _zero_fill(zbuf, 16)
        one = jnp.ones((16,), jnp.float32)

        def fill1(i, carry):
            ones_v[i, pl.ds(0, 16)] = one
            return carry

        lax.fori_loop(0, CHUNK, fill1, 0)
        sl = pl.ds(sid * ROWS_PER_TILE, ROWS_PER_TILE)
        pltpu.sync_copy(zbuf, acc.at[sl])
        _load_edges(ei_hbm, 1, wid, didx)
        plsc.subcore_barrier()

        def body(t, carry):
            @pl.when(t > 0)
            def _():
                for b in range(GROUP):
                    pltpu.make_async_copy(ones_v, acc.at[dbufs[b]], sems[b]).wait()

            for b in range(GROUP):
                _copy_idx(didx, t * GROUP + b, dbufs[b])
                pltpu.async_copy(ones_v, acc.at[dbufs[b]], sems[b], add=True)
            return carry

        lax.fori_loop(0, NGROUPS, body, 0)
        for b in range(GROUP):
            pltpu.make_async_copy(ones_v, acc.at[dbufs[b]], sems[b]).wait()
        plsc.subcore_barrier()
        pltpu.sync_copy(acc.at[sl], out_hbm.at[cid, sl])

    return k


@functools.lru_cache(maxsize=None)
def _make_sc_segsum(d):
    @functools.partial(
        pl.kernel,
        out_type=jax.ShapeDtypeStruct((NC, NPAD, d), jnp.float32),
        mesh=_sc_mesh(),
        scratch_types=(
            [pltpu.VMEM((EPT,), jnp.int32),
             pltpu.VMEM((EPT,), jnp.int32),
             pltpu.VMEM((ROWS_PER_TILE, d), jnp.float32),
             pltpu.VMEM_SHARED((NPAD, d), jnp.float32)]
            + [pltpu.VMEM((CHUNK, d), jnp.float32) for _ in range(GROUP)]
            + [pltpu.VMEM((CHUNK,), jnp.int32) for _ in range(GROUP)]
            + [pltpu.SemaphoreType.DMA for _ in range(2 * GROUP)]),
        compiler_params=pltpu.CompilerParams(use_tc_tiling_on_sc=False))
    def k(table_hbm, ei_hbm, out_hbm, sidx, didx, zbuf, acc, *ring):
        rows = ring[:GROUP]
        dbufs = ring[GROUP:2 * GROUP]
        gsems = ring[2 * GROUP:3 * GROUP]
        ssems = ring[3 * GROUP:]
        cid = lax.axis_index("c")
        sid = lax.axis_index("s")
        wid = cid * NS + sid
        _zero_fill(zbuf, d)
        sl = pl.ds(sid * ROWS_PER_TILE, ROWS_PER_TILE)
        pltpu.sync_copy(zbuf, acc.at[sl])
        _load_edges(ei_hbm, 0, wid, sidx)
        _load_edges(ei_hbm, 1, wid, didx)
        plsc.subcore_barrier()

        def gather(c, b):
            pltpu.async_copy(table_hbm.at[sidx.at[pl.ds(c * CHUNK, CHUNK)]],
                             rows[b], gsems[b])

        def gwait(c, b):
            pltpu.make_async_copy(table_hbm.at[sidx.at[pl.ds(c * CHUNK, CHUNK)]],
                                  rows[b], gsems[b]).wait()

        def scat(b):
            pltpu.async_copy(rows[b], acc.at[dbufs[b]], ssems[b], add=True)

        def swait(b):
            pltpu.make_async_copy(rows[b], acc.at[dbufs[b]], ssems[b]).wait()

        def body(t, carry):
            base = t * GROUP
            for b in range(NBUF):
                gather(base + b, b)

            @pl.when(t > 0)
            def _():
                for b in range(NBUF, GROUP):
                    swait(b)

            for b in range(NBUF):
                _copy_idx(didx, base + b, dbufs[b])
                gwait(base + b, b)
                scat(b)
            for b in range(NBUF, GROUP):
                gather(base + b, b)
            for b in range(NBUF):
                swait(b)
            for b in range(NBUF, GROUP):
                _copy_idx(didx, base + b, dbufs[b])
                gwait(base + b, b)
                scat(b)
            return carry

        lax.fori_loop(0, NGROUPS, body, 0)
        for b in range(NBUF, GROUP):
            swait(b)
        plsc.subcore_barrier()
        pltpu.sync_copy(acc.at[sl], out_hbm.at[cid, sl])

    return k


BR = 2000
BA = 400


def _full(shape):
    return pl.BlockSpec(shape, lambda i: tuple(0 for _ in shape))


def _rows(d):
    return pl.BlockSpec((BR, d), lambda i: (i, 0))


def _tc_encode(x, gc1_W, enc_W1, enc_b1, enc_W2, enc_b2,
               dec_W1, dec_b1, dec_W2, dec_b2):
    def body(x_ref, w1, ew1, eb1, ew2, eb2, dw1, db1, dw2, db2,
             h1_ref, zx_ref, fr_ref):
        x = x_ref[...]
        h1_ref[...] = _dot(x, w1[...])
        e1 = jnp.maximum(_dot(x, ew1[...]) + eb1[...], 0.0)
        zx = _dot(e1, ew2[...]) + eb2[...]
        zx_ref[...] = zx
        dh = _dot(zx, dw1[...]) + db1[...]
        fr_ref[...] = _sigmoid(_dot(dh, dw2[...]) + db2[...])

    return pl.pallas_call(
        body,
        grid=(N // BR,),
        in_specs=[_rows(F_IN),
                  _full((F_IN, NHID)), _full((F_IN, NHID)), _full((1, NHID)),
                  _full((NHID, LAT)), _full((1, LAT)),
                  _full((LAT, NHID)), _full((1, NHID)),
                  _full((NHID, F_IN)), _full((1, F_IN))],
        out_specs=[_rows(NHID), _rows(LAT), _rows(F_IN)],
        out_shape=[jax.ShapeDtypeStruct((N, NHID), jnp.float32),
                   jax.ShapeDtypeStruct((N, LAT), jnp.float32),
                   jax.ShapeDtypeStruct((N, F_IN), jnp.float32)],
    )(x, gc1_W, enc_W1, enc_b1, enc_W2, enc_b2, dec_W1, dec_b1, dec_W2, dec_b2)


def _deg_spec():
    return pl.BlockSpec((NC, BR, 16), lambda i: (0, i, 0))


def _dinv_of(dg):
    deg = 1.0 + dg[0, :, 0] + dg[1, :, 0]
    return lax.rsqrt(deg)[:, None]


def _tc_g1(deg2, h1):
    def body(dg, h1_ref, g1_ref):
        g1_ref[...] = h1_ref[...] * _dinv_of(dg[...])

    return pl.pallas_call(
        body,
        grid=(N // BR,),
        in_specs=[_deg_spec(), _rows(NHID)],
        out_specs=_rows(NHID),
        out_shape=jax.ShapeDtypeStruct((N, NHID), jnp.float32),
    )(deg2, h1)


def _tc_mid(agg1, h1, deg2, gc1_b, gc2_W):
    def body(ag, h1_ref, dg, b1, w2, h2_ref, g2_ref):
        dinv = _dinv_of(dg[...])
        a = ag[0] + ag[1]
        hmid = jnp.maximum(a * dinv + h1_ref[...] * (dinv * dinv) + b1[...], 0.0)
        h2 = _dot(hmid, w2[...])
        h2_ref[...] = h2
        g2_ref[...] = h2 * dinv

    return pl.pallas_call(
        body,
        grid=(N // BR,),
        in_specs=[pl.BlockSpec((NC, BR, NHID), lambda i: (0, i, 0)),
                  _rows(NHID), _deg_spec(), _full((1, NHID)),
                  _full((NHID, LAT))],
        out_specs=[_rows(LAT), _rows(LAT)],
        out_shape=[jax.ShapeDtypeStruct((N, LAT), jnp.float32),
                   jax.ShapeDtypeStruct((N, LAT), jnp.float32)],
    )(agg1, h1, deg2, gc1_b, gc2_W)


def _tc_head(agg2, h2, deg2, gc2_b, zx, pl_W, pl_b):
    def body(ag, h2_ref, dg, b2, zx_ref, plw, plb, z_ref, pred_ref):
        dinv = _dinv_of(dg[...])
        za = jnp.maximum((ag[0] + ag[1]) * dinv
                         + h2_ref[...] * (dinv * dinv) + b2[...], 0.0)
        z = jnp.concatenate([za, zx_ref[...]], axis=1)
        z_ref[...] = z
        lg = _dot(z, plw[...]) + plb[...]
        m = jnp.max(lg, axis=1, keepdims=True)
        e = jnp.exp(lg - m)
        pred_ref[...] = e / jnp.sum(e, axis=1, keepdims=True)

    return pl.pallas_call(
        body,
        grid=(N // BR,),
        in_specs=[pl.BlockSpec((NC, BR, LAT), lambda i: (0, i, 0)),
                  _rows(LAT), _deg_spec(), _full((1, LAT)), _rows(LAT),
                  _full((2 * LAT, NCLS)), _full((1, NCLS))],
        out_specs=[_rows(2 * LAT), _rows(NCLS)],
        out_shape=[jax.ShapeDtypeStruct((N, 2 * LAT), jnp.float32),
                   jax.ShapeDtypeStruct((N, NCLS), jnp.float32)],
    )(agg2, h2, deg2, gc2_b, zx, pl_W, pl_b)


def _tc_adj(z):
    def body(zf, out_ref):
        i = pl.program_id(0)
        zi = zf[pl.ds(i * BA, BA), :]
        out_ref[...] = _sigmoid(lax.dot_general(
            zi, zf[...], (((1,), (1,)), ((), ())),
            preferred_element_type=jnp.float32))

    return pl.pallas_call(
        body,
        grid=(N // BA,),
        in_specs=[pl.BlockSpec((N, 2 * LAT), lambda i: (0, 0))],
        out_specs=pl.BlockSpec((BA, N), lambda i: (i, 0)),
        out_shape=jax.ShapeDtypeStruct((N, N), jnp.float32),
    )(z)


def kernel(features, edge_index, gc1_W, gc1_b, gc2_W, gc2_b,
           enc_W1, enc_b1, enc_W2, enc_b2,
           dec_W1, dec_b1, dec_W2, dec_b2, pl_W, pl_b):
    b = lambda v: v.reshape(1, -1)

    deg2 = _sc_degree_kernel()(edge_index)
    h1, zx, fr = _tc_encode(features, gc1_W, enc_W1, b(enc_b1), enc_W2,
                            b(enc_b2), dec_W1, b(dec_b1), dec_W2, b(dec_b2))
    g1 = _tc_g1(deg2, h1)
    g1p = jnp.pad(g1, ((0, NPAD - N), (0, 0)))
    agg1 = _make_sc_segsum(NHID)(g1p, edge_index)
    h2, g2 = _tc_mid(agg1, h1, deg2, b(gc1_b), gc2_W)
    g2p = jnp.pad(g2, ((0, NPAD - N), (0, 0)))
    agg2 = _make_sc_segsum(LAT)(g2p, edge_index)
    z, pred = _tc_head(agg2, h2, deg2, b(gc2_b), zx, pl_W, b(pl_b))
    adj = _tc_adj(z)
    return adj, fr, pred, z


# speedup vs baseline: 25.3931x; 1.0002x over previous
import functools

import jax
import jax.numpy as jnp
from jax import lax
from jax.experimental import pallas as pl
from jax.experimental.pallas import tpu as pltpu
from jax.experimental.pallas import tpu_sc as plsc

N = 10000
NPAD = 10112
F_IN = 128
NHID = 32
LAT = 16
NCLS = 16
E = 320000

NC = 2
NS = 16
NW = NC * NS
CHUNK = 128
NBUF = 4
GROUP = 2 * NBUF
NCHUNKS = 80
NGROUPS = NCHUNKS // GROUP
EPT = NCHUNKS * CHUNK
EPW = E // NW
ROWS_PER_TILE = NPAD // NS

def _dot(a, b):
    return lax.dot_general(a, b, (((1,), (0,)), ((), ())),
                           preferred_element_type=jnp.float32)


def _sigmoid(x):
    return lax.logistic(x)


def _sc_mesh():
    return plsc.VectorSubcoreMesh(core_axis_name="c", subcore_axis_name="s",
                                  num_cores=NC, num_subcores=NS)


def _zero_fill(buf, d):
    z = jnp.zeros((16,), jnp.float32)

    def body(i, carry):
        for c in range(d // 16):
            buf[i, pl.ds(c * 16, 16)] = z
        return carry

    lax.fori_loop(0, buf.shape[0], body, 0)


def _copy_idx(idxbuf, c, dbuf):
    for kk in range(CHUNK // 16):
        dbuf[pl.ds(kk * 16, 16)] = idxbuf[pl.ds(c * CHUNK + kk * 16, 16)]


def _load_edges(ei_hbm, row, wid, idxbuf):
    pltpu.sync_copy(ei_hbm.at[row, pl.ds(wid * EPW, EPW)],
                    idxbuf.at[pl.ds(0, EPW)])
    lanes = lax.iota(jnp.int32, 16)
    for t in range(EPW, EPT, 16):
        idxbuf[pl.ds(t, 16)] = N + ((t - EPW + lanes) % (NPAD - N))


@functools.lru_cache(maxsize=None)
def _sc_degree_kernel():
    @functools.partial(
        pl.kernel,
        out_type=jax.ShapeDtypeStruct((NC, NPAD, 16), jnp.float32),
        mesh=_sc_mesh(),
        scratch_types=(
            [pltpu.VMEM((EPT,), jnp.int32),
             pltpu.VMEM((CHUNK, 16), jnp.float32),
             pltpu.VMEM((ROWS_PER_TILE, 16), jnp.float32),
             pltpu.VMEM_SHARED((NPAD, 16), jnp.float32)]
            + [pltpu.VMEM((CHUNK,), jnp.int32) for _ in range(GROUP)]
            + [pltpu.SemaphoreType.DMA for _ in range(GROUP)]),
        compiler_params=pltpu.CompilerParams(use_tc_tiling_on_sc=False))
    def k(ei_hbm, out_hbm, didx, ones_v, zbuf, acc, *ring):
        dbufs = ring[:GROUP]
        sems = ring[GROUP:]
        cid = lax.axis_index("c")
        sid = lax.axis_index("s")
        wid = cid * NS + sid
        _zero_fill(zbuf, 16)
        one = jnp.ones((16,), jnp.float32)

        def fill1(i, carry):
            ones_v[i, pl.ds(0, 16)] = one
            return carry

        lax.fori_loop(0, CHUNK, fill1, 0)
        sl = pl.ds(sid * ROWS_PER_TILE, ROWS_PER_TILE)
        pltpu.sync_copy(zbuf, acc.at[sl])
        _load_edges(ei_hbm, 1, wid, didx)
        plsc.subcore_barrier()

        def body(t, carry):
            @pl.when(t > 0)
            def _():
                for b in range(GROUP):
                    pltpu.make_async_copy(ones_v, acc.at[dbufs[b]], sems[b]).wait()

            for b in range(GROUP):
                _copy_idx(didx, t * GROUP + b, dbufs[b])
                pltpu.async_copy(ones_v, acc.at[dbufs[b]], sems[b], add=True)
            return carry

        lax.fori_loop(0, NGROUPS, body, 0)
        for b in range(GROUP):
            pltpu.make_async_copy(ones_v, acc.at[dbufs[b]], sems[b]).wait()
        plsc.subcore_barrier()
        pltpu.sync_copy(acc.at[sl], out_hbm.at[cid, sl])

    return k


@functools.lru_cache(maxsize=None)
def _make_sc_segsum(d):
    @functools.partial(
        pl.kernel,
        out_type=jax.ShapeDtypeStruct((NC, NPAD, d), jnp.float32),
        mesh=_sc_mesh(),
        scratch_types=(
            [pltpu.VMEM((EPT,), jnp.int32),
             pltpu.VMEM((EPT,), jnp.int32),
             pltpu.VMEM((ROWS_PER_TILE, d), jnp.float32),
             pltpu.VMEM_SHARED((NPAD, d), jnp.float32)]
            + [pltpu.VMEM((CHUNK, d), jnp.float32) for _ in range(GROUP)]
            + [pltpu.VMEM((CHUNK,), jnp.int32) for _ in range(GROUP)]
            + [pltpu.SemaphoreType.DMA for _ in range(2 * GROUP)]),
        compiler_params=pltpu.CompilerParams(use_tc_tiling_on_sc=False))
    def k(table_hbm, ei_hbm, out_hbm, sidx, didx, zbuf, acc, *ring):
        rows = ring[:GROUP]
        dbufs = ring[GROUP:2 * GROUP]
        gsems = ring[2 * GROUP:3 * GROUP]
        ssems = ring[3 * GROUP:]
        cid = lax.axis_index("c")
        sid = lax.axis_index("s")
        wid = cid * NS + sid
        _zero_fill(zbuf, d)
        sl = pl.ds(sid * ROWS_PER_TILE, ROWS_PER_TILE)
        pltpu.sync_copy(zbuf, acc.at[sl])
        _load_edges(ei_hbm, 0, wid, sidx)
        _load_edges(ei_hbm, 1, wid, didx)
        plsc.subcore_barrier()

        def gather(c, b):
            pltpu.async_copy(table_hbm.at[sidx.at[pl.ds(c * CHUNK, CHUNK)]],
                             rows[b], gsems[b])

        def gwait(c, b):
            pltpu.make_async_copy(table_hbm.at[sidx.at[pl.ds(c * CHUNK, CHUNK)]],
                                  rows[b], gsems[b]).wait()

        def scat(b):
            pltpu.async_copy(rows[b], acc.at[dbufs[b]], ssems[b], add=True)

        def swait(b):
            pltpu.make_async_copy(rows[b], acc.at[dbufs[b]], ssems[b]).wait()

        def body(t, carry):
            base = t * GROUP
            for b in range(NBUF):
                gather(base + b, b)

            @pl.when(t > 0)
            def _():
                for b in range(NBUF, GROUP):
                    swait(b)

            for b in range(NBUF):
                _copy_idx(didx, base + b, dbufs[b])
                gwait(base + b, b)
                scat(b)
            for b in range(NBUF, GROUP):
                gather(base + b, b)
            for b in range(NBUF):
                swait(b)
            for b in range(NBUF, GROUP):
                _copy_idx(didx, base + b, dbufs[b])
                gwait(base + b, b)
                scat(b)
            return carry

        lax.fori_loop(0, NGROUPS, body, 0)
        for b in range(NBUF, GROUP):
            swait(b)
        plsc.subcore_barrier()
        pltpu.sync_copy(acc.at[sl], out_hbm.at[cid, sl])

    return k


BR = 2000
BA = 400


def _full(shape):
    return pl.BlockSpec(shape, lambda i: tuple(0 for _ in shape))


def _rows(d):
    return pl.BlockSpec((BR, d), lambda i: (i, 0))


def _tc_encode(x, gc1_W, enc_W1, enc_b1, enc_W2, enc_b2,
               dec_W1, dec_b1, dec_W2, dec_b2):
    def body(x_ref, w1, ew1, eb1, ew2, eb2, dw1, db1, dw2, db2,
             h1_ref, zx_ref, fr_ref):
        x = x_ref[...]
        h1_ref[...] = _dot(x, w1[...])
        e1 = jnp.maximum(_dot(x, ew1[...]) + eb1[...], 0.0)
        zx = _dot(e1, ew2[...]) + eb2[...]
        zx_ref[...] = zx
        dh = _dot(zx, dw1[...]) + db1[...]
        fr_ref[...] = _sigmoid(_dot(dh, dw2[...]) + db2[...])

    return pl.pallas_call(
        body,
        grid=(N // BR,),
        in_specs=[_rows(F_IN),
                  _full((F_IN, NHID)), _full((F_IN, NHID)), _full((1, NHID)),
                  _full((NHID, LAT)), _full((1, LAT)),
                  _full((LAT, NHID)), _full((1, NHID)),
                  _full((NHID, F_IN)), _full((1, F_IN))],
        out_specs=[_rows(NHID), _rows(LAT), _rows(F_IN)],
        out_shape=[jax.ShapeDtypeStruct((N, NHID), jnp.float32),
                   jax.ShapeDtypeStruct((N, LAT), jnp.float32),
                   jax.ShapeDtypeStruct((N, F_IN), jnp.float32)],
    )(x, gc1_W, enc_W1, enc_b1, enc_W2, enc_b2, dec_W1, dec_b1, dec_W2, dec_b2)


def _deg_spec():
    return pl.BlockSpec((NC, BR, 16), lambda i: (0, i, 0))


def _dinv_of(dg):
    deg = 1.0 + dg[0, :, 0] + dg[1, :, 0]
    return lax.rsqrt(deg)[:, None]


def _tc_g1(deg2, h1):
    def body(dg, h1_ref, g1_ref):
        g1_ref[...] = h1_ref[...] * _dinv_of(dg[...])

    return pl.pallas_call(
        body,
        grid=(N // BR,),
        in_specs=[_deg_spec(), _rows(NHID)],
        out_specs=_rows(NHID),
        out_shape=jax.ShapeDtypeStruct((N, NHID), jnp.float32),
    )(deg2, h1)


def _tc_mid(agg1, h1, deg2, gc1_b, gc2_W):
    def body(ag, h1_ref, dg, b1, w2, h2_ref, g2_ref):
        dinv = _dinv_of(dg[...])
        a = ag[0] + ag[1]
        hmid = jnp.maximum(a * dinv + h1_ref[...] * (dinv * dinv) + b1[...], 0.0)
        h2 = _dot(hmid, w2[...])
        h2_ref[...] = h2
        g2_ref[...] = h2 * dinv

    return pl.pallas_call(
        body,
        grid=(N // BR,),
        in_specs=[pl.BlockSpec((NC, BR, NHID), lambda i: (0, i, 0)),
                  _rows(NHID), _deg_spec(), _full((1, NHID)),
                  _full((NHID, LAT))],
        out_specs=[_rows(LAT), _rows(LAT)],
        out_shape=[jax.ShapeDtypeStruct((N, LAT), jnp.float32),
                   jax.ShapeDtypeStruct((N, LAT), jnp.float32)],
    )(agg1, h1, deg2, gc1_b, gc2_W)


def _tc_head(agg2, h2, deg2, gc2_b, zx, pl_W, pl_b):
    def body(ag, h2_ref, dg, b2, zx_ref, plw, plb, z_ref, z2_ref, pred_ref):
        dinv = _dinv_of(dg[...])
        za = jnp.maximum((ag[0] + ag[1]) * dinv
                         + h2_ref[...] * (dinv * dinv) + b2[...], 0.0)
        z = jnp.concatenate([za, zx_ref[...]], axis=1)
        z_ref[...] = z
        z2_ref[...] = z
        lg = _dot(z, plw[...]) + plb[...]
        m = jnp.max(lg, axis=1, keepdims=True)
        e = jnp.exp(lg - m)
        pred_ref[...] = e / jnp.sum(e, axis=1, keepdims=True)

    return pl.pallas_call(
        body,
        grid=(N // BR,),
        in_specs=[pl.BlockSpec((NC, BR, LAT), lambda i: (0, i, 0)),
                  _rows(LAT), _deg_spec(), _full((1, LAT)), _rows(LAT),
                  _full((2 * LAT, NCLS)), _full((1, NCLS))],
        out_specs=[_rows(2 * LAT), _rows(2 * LAT), _rows(NCLS)],
        out_shape=[jax.ShapeDtypeStruct((N, 2 * LAT), jnp.float32),
                   jax.ShapeDtypeStruct((N, 2 * LAT), jnp.float32),
                   jax.ShapeDtypeStruct((N, NCLS), jnp.float32)],
    )(agg2, h2, deg2, gc2_b, zx, pl_W, pl_b)


def _tc_adj(z):
    def body(zf, out_ref):
        i = pl.program_id(0)
        zi = zf[pl.ds(i * BA, BA), :]
        out_ref[...] = _sigmoid(lax.dot_general(
            zi, zf[...], (((1,), (1,)), ((), ())),
            preferred_element_type=jnp.float32))

    return pl.pallas_call(
        body,
        grid=(N // BA,),
        in_specs=[pl.BlockSpec((N, 2 * LAT), lambda i: (0, 0))],
        out_specs=pl.BlockSpec((BA, N), lambda i: (i, 0)),
        out_shape=jax.ShapeDtypeStruct((N, N), jnp.float32),
    )(z)


def kernel(features, edge_index, gc1_W, gc1_b, gc2_W, gc2_b,
           enc_W1, enc_b1, enc_W2, enc_b2,
           dec_W1, dec_b1, dec_W2, dec_b2, pl_W, pl_b):
    b = lambda v: v.reshape(1, -1)

    deg2 = _sc_degree_kernel()(edge_index)
    h1, zx, fr = _tc_encode(features, gc1_W, enc_W1, b(enc_b1), enc_W2,
                            b(enc_b2), dec_W1, b(dec_b1), dec_W2, b(dec_b2))
    g1 = _tc_g1(deg2, h1)
    g1p = jnp.pad(g1, ((0, NPAD - N), (0, 0)))
    agg1 = _make_sc_segsum(NHID)(g1p, edge_index)
    h2, g2 = _tc_mid(agg1, h1, deg2, b(gc1_b), gc2_W)
    g2p = jnp.pad(g2, ((0, NPAD - N), (0, 0)))
    agg2 = _make_sc_segsum(LAT)(g2p, edge_index)
    z, z2, pred = _tc_head(agg2, h2, deg2, b(gc2_b), zx, pl_W, b(pl_b))
    adj = _tc_adj(z2)
    return adj, fr, pred, z
